# baseline (speedup 1.0000x reference)
import functools

import jax
import jax.numpy as jnp
from jax import lax
from jax.experimental import pallas as pl
from jax.experimental.pallas import tpu as pltpu
from jax.experimental.pallas import tpu_sc as plsc

NC, NS = 2, 16
NW = NC * NS
B = 1024 * 512
D = 128
V = 28
CH = 256
CPW = B // (NW * CH)
NBUF = 3
L = 16


def _sc_embed_body(x_hbm, table_hbm, out_hbm, idx_v, table_v, rows_v, wsem):
    wid = lax.axis_index("s") * NC + lax.axis_index("c")
    pltpu.sync_copy(table_hbm, table_v)
    pltpu.sync_copy(x_hbm.at[wid], idx_v)
    base0 = wid * CPW * CH

    def build(c, b):
        @plsc.parallel_loop(0, CH // L, unroll=2)
        def grp(g):
            vec = idx_v[c, pl.ds(g * L, L)]
            for l in range(L):
                row = table_v.at[vec[l]]
                i = g * L + l
                for k in range(D // L):
                    rows_v[b, i, pl.ds(k * L, L)] = row[pl.ds(k * L, L)]

    def start_write(c, b):
        pltpu.async_copy(rows_v.at[b], out_hbm.at[pl.ds(base0 + c * CH, CH)], wsem)

    def wait_write(c, b):
        pltpu.make_async_copy(
            rows_v.at[b], out_hbm.at[pl.ds(base0 + c * CH, CH)], wsem
        ).wait()

    def step(c, carry):
        b = lax.rem(c, NBUF)

        @pl.when(c >= NBUF)
        def _():
            wait_write(c - NBUF, b)

        build(c, b)
        start_write(c, b)
        return carry

    lax.fori_loop(0, CPW, step, 0)

    for c in range(CPW - NBUF, CPW):
        wait_write(c, c % NBUF)


_sc_embed = functools.partial(
    pl.kernel,
    out_type=jax.ShapeDtypeStruct((B, D), jnp.float32),
    mesh=plsc.VectorSubcoreMesh(
        core_axis_name="c", subcore_axis_name="s", num_cores=NC, num_subcores=NS
    ),
    scratch_types=[
        pltpu.VMEM((CPW, CH), jnp.int32),
        pltpu.VMEM((V, D), jnp.float32),
        pltpu.VMEM((NBUF, CH, D), jnp.float32),
        pltpu.SemaphoreType.DMA,
    ],
)(_sc_embed_body)


@jax.jit
def kernel(x, table):
    xw = x.astype(jnp.int32).reshape(NW, CPW, CH)
    out = _sc_embed(xw, table)
    return out.reshape(1024, 512, 128)


# speedup vs baseline: 1.4300x; 1.4300x over previous
import functools

import jax
import jax.numpy as jnp
from jax import lax
from jax.experimental import pallas as pl
from jax.experimental.pallas import tpu as pltpu
from jax.experimental.pallas import tpu_sc as plsc

NC, NS = 2, 16
NW = NC * NS
B = 1024 * 512
D = 128
V = 28
CH = 256
CPW = B // (NW * CH)
NBUF = 3
L = 16


def _sc_embed_body(x_hbm, table_hbm, out_hbm, idx_v, table_v, rows_v, wsem):
    wid = lax.axis_index("s") * NC + lax.axis_index("c")
    pltpu.sync_copy(table_hbm, table_v)
    pltpu.sync_copy(x_hbm.at[wid], idx_v)
    base0 = wid * CPW * CH

    def build(c, b):
        @plsc.parallel_loop(0, CH // L, unroll=1)
        def grp(g):
            vec = idx_v[c, pl.ds(g * L, L)]
            for l in range(L):
                row = table_v.at[vec[l]]
                i = g * L + l
                for k in range(D // L):
                    rows_v[b, i, pl.ds(k * L, L)] = row[pl.ds(k * L, L)]

    def start_write(c, b):
        pltpu.async_copy(rows_v.at[b], out_hbm.at[pl.ds(base0 + c * CH, CH)], wsem)

    def wait_write(c, b):
        pltpu.make_async_copy(
            rows_v.at[b], out_hbm.at[pl.ds(base0 + c * CH, CH)], wsem
        ).wait()

    def step(c, carry):
        b = lax.rem(c, NBUF)

        @pl.when(c >= NBUF)
        def _():
            wait_write(c - NBUF, b)

        build(c, b)
        start_write(c, b)
        return carry

    lax.fori_loop(0, CPW, step, 0)

    for c in range(CPW - NBUF, CPW):
        wait_write(c, c % NBUF)


_sc_embed = functools.partial(
    pl.kernel,
    out_type=jax.ShapeDtypeStruct((B, D), jnp.float32),
    mesh=plsc.VectorSubcoreMesh(
        core_axis_name="c", subcore_axis_name="s", num_cores=NC, num_subcores=NS
    ),
    scratch_types=[
        pltpu.VMEM((CPW, CH), jnp.int32),
        pltpu.VMEM((V, D), jnp.float32),
        pltpu.VMEM((NBUF, CH, D), jnp.float32),
        pltpu.SemaphoreType.DMA,
    ],
)(_sc_embed_body)


@jax.jit
def kernel(x, table):
    xw = x.astype(jnp.int32).reshape(NW, CPW, CH)
    out = _sc_embed(xw, table)
    return out.reshape(1024, 512, 128)


# speedup vs baseline: 1.8254x; 1.2765x over previous
import functools

import jax
import jax.numpy as jnp
from jax import lax
from jax.experimental import pallas as pl
from jax.experimental.pallas import tpu as pltpu
from jax.experimental.pallas import tpu_sc as plsc

NC, NS = 2, 16
NW = NC * NS
B = 1024 * 512
D = 128
V = 28
RPW = B // NW
L = 16
WIN = 64


def _sc_embed_body(x_hbm, table_hbm, out_hbm, idx_v, table_v, wsem):
    wid = lax.axis_index("s") * NC + lax.axis_index("c")
    pltpu.sync_copy(table_hbm, table_v)
    pltpu.sync_copy(x_hbm.at[wid], idx_v)
    base0 = wid * RPW

    def start_row(i, s):
        pltpu.async_copy(table_v.at[pl.ds(s, 1)], out_hbm.at[pl.ds(base0 + i, 1)], wsem)

    def drain_one():
        pltpu.make_async_copy(
            table_v.at[pl.ds(0, 1)], out_hbm.at[pl.ds(base0, 1)], wsem
        ).wait()

    def grp(g, carry):
        vec = idx_v[pl.ds(g * L, L)]

        @pl.when(g >= WIN // L)
        def _():
            for _ in range(L):
                drain_one()

        for l in range(L):
            start_row(g * L + l, vec[l])
        return carry

    lax.fori_loop(0, RPW // L, grp, 0)

    for _ in range(WIN):
        drain_one()


_sc_embed = functools.partial(
    pl.kernel,
    out_type=jax.ShapeDtypeStruct((B, D), jnp.float32),
    mesh=plsc.VectorSubcoreMesh(
        core_axis_name="c", subcore_axis_name="s", num_cores=NC, num_subcores=NS
    ),
    scratch_types=[
        pltpu.VMEM((RPW,), jnp.int32),
        pltpu.VMEM((V, D), jnp.float32),
        pltpu.SemaphoreType.DMA,
    ],
)(_sc_embed_body)


@jax.jit
def kernel(x, table):
    xw = x.astype(jnp.int32).reshape(NW, RPW)
    out = _sc_embed(xw, table)
    return out.reshape(1024, 512, 128)


# speedup vs baseline: 1.8254x; 1.0000x over previous
import functools

import jax
import jax.numpy as jnp
from jax import lax
from jax.experimental import pallas as pl
from jax.experimental.pallas import tpu as pltpu
from jax.experimental.pallas import tpu_sc as plsc

NC, NS = 2, 16
NW = NC * NS
B = 1024 * 512
D = 128
V = 28
RPW = B // NW
L = 16
WIN = 128


def _sc_embed_body(x_hbm, table_hbm, out_hbm, idx_v, table_v, wsem):
    wid = lax.axis_index("s") * NC + lax.axis_index("c")
    pltpu.sync_copy(table_hbm, table_v)
    pltpu.sync_copy(x_hbm.at[wid], idx_v)
    base0 = wid * RPW

    def start_row(i, s):
        pltpu.async_copy(table_v.at[pl.ds(s, 1)], out_hbm.at[pl.ds(base0 + i, 1)], wsem)

    def drain_one():
        pltpu.make_async_copy(
            table_v.at[pl.ds(0, 1)], out_hbm.at[pl.ds(base0, 1)], wsem
        ).wait()

    def grp(g, carry):
        vec = idx_v[pl.ds(g * L, L)]

        @pl.when(g >= WIN // L)
        def _():
            for _ in range(L):
                drain_one()

        for l in range(L):
            start_row(g * L + l, vec[l])
        return carry

    lax.fori_loop(0, RPW // L, grp, 0)

    for _ in range(WIN):
        drain_one()


_sc_embed = functools.partial(
    pl.kernel,
    out_type=jax.ShapeDtypeStruct((B, D), jnp.float32),
    mesh=plsc.VectorSubcoreMesh(
        core_axis_name="c", subcore_axis_name="s", num_cores=NC, num_subcores=NS
    ),
    scratch_types=[
        pltpu.VMEM((RPW,), jnp.int32),
        pltpu.VMEM((V, D), jnp.float32),
        pltpu.SemaphoreType.DMA,
    ],
)(_sc_embed_body)


@jax.jit
def kernel(x, table):
    xw = x.astype(jnp.int32).reshape(NW, RPW)
    out = _sc_embed(xw, table)
    return out.reshape(1024, 512, 128)
